# baseline (speedup 1.0000x reference)
import jax
import jax.numpy as jnp
from jax.experimental import pallas as pl
from jax.experimental.pallas import tpu as pltpu

B = 16; P = 16; NP = 128; EP = 1024
NPAT = 16; EPAT = 80
D_ORIG = 256; H1 = 64; HEADS = 2; OUT1 = 32; NH = 128
EPS = 1e-5
INV = 1.0 / (1.0 + EPS) ** 0.5
NEG = -1e30


def _lrelu(x):
    return jnp.maximum(x, 0.2 * x)


def _dot(a, b):
    return jax.lax.dot_general(a, b, (((1,), (0,)), ((), ())),
                               preferred_element_type=jnp.float32)


def _dot_t(a, b):
    return jax.lax.dot_general(a, b, (((1,), (1,)), ((), ())),
                               preferred_element_type=jnp.float32)


def _bdot(a, b, ca, cb):
    return jax.lax.dot_general(a, b, (((ca,), (cb,)), ((0,), (0,))),
                               preferred_element_type=jnp.float32)


def _ct_kernel(src_ref, dst_ref, lct_ref):
    iota_e = jax.lax.broadcasted_iota(jnp.int32, (NP, EP), 0)
    ohs = (src_ref[:, :] == iota_e).astype(jnp.float32)
    ohd = (dst_ref[:, :] == iota_e).astype(jnp.float32)
    ct = _dot_t(ohd, ohs)
    lct_ref[:, :] = jnp.where(ct > 0.5, jnp.log(jnp.maximum(ct, 0.5)), NEG)


def _plane_kernel(lct_ref, pf_col_ref, pf_row_ref,
                  g1_fc_ref, g1_al_ref, g1_ar_ref, g1_res_ref, g1_b_ref,
                  bn1_g_ref, bn1_b_ref,
                  g2_fc_ref, g2_al_ref, g2_ar_ref, g2_res_ref, g2_b_ref,
                  bn2_g_ref, bn2_b_ref,
                  dec_w1_ref, dec_b1_ref, dec_bng_ref, dec_bnb_ref,
                  dec_w2_ref, dec_b2_ref,
                  rep_ref, rloss_ref):
    lct3 = lct_ref[:, :][None, :, :]

    h0c = pf_col_ref[:, 0, :, :]
    h0r = pf_row_ref[:, 0, :, :]

    fc1 = g1_fc_ref[0]
    al1 = g1_al_ref[0]
    ar1 = g1_ar_ref[0]
    prod_l = fc1 * al1
    prod_r = fc1 * ar1
    cl0 = jnp.sum(prod_l[:, :H1]); cl1 = jnp.sum(prod_l[:, H1:])
    cr0 = jnp.sum(prod_r[:, :H1]); cr1 = jnp.sum(prod_r[:, H1:])
    hmax = jnp.max(h0r, axis=2, keepdims=True)
    hmin = jnp.min(h0r, axis=2, keepdims=True)
    ones_c = h0c * 0.0 + 1.0
    h0_aug = jnp.concatenate([h0c, ones_c], axis=2)

    def gat1_head(cl, cr):
        elmax = jnp.maximum(cl * hmax, cl * hmin)
        shift = _lrelu(elmax + cr * h0c)
        ee = jnp.exp(_lrelu(cl * h0r + cr * h0c) + lct3 - shift)
        sums = _bdot(ee, h0_aug, 2, 1)
        den = sums[:, :, 1:2]
        return jnp.where(den > 0.0, sums[:, :, 0:1] / den, 0.0)

    s0 = gat1_head(cl0, cr0)
    s1 = gat1_head(cl1, cr1)
    lane = jax.lax.broadcasted_iota(jnp.int32, (1, HEADS * H1), 1)
    hsel0 = (lane < H1).astype(jnp.float32)
    w3 = jnp.concatenate([fc1 * hsel0, fc1 * (1.0 - hsel0), g1_res_ref[0]],
                         axis=0)
    lhs = jnp.concatenate([s0, s1, h0c], axis=2).reshape(B * NP, 3)
    rst1 = _dot(lhs, w3) + g1_b_ref[0]
    h1f = jnp.maximum(rst1 * (INV * bn1_g_ref[0]) + bn1_b_ref[0], 0.0)

    feat2 = _dot(h1f, g2_fc_ref[0]).reshape(B, NP, OUT1)
    ones2 = feat2[:, :, 0:1] * 0.0 + 1.0
    feat2_aug = jnp.concatenate([feat2, ones2], axis=2)
    al2 = jnp.broadcast_to(g2_al_ref[0][None], (B, 1, OUT1))
    ar2 = jnp.broadcast_to(g2_ar_ref[0][None], (B, 1, OUT1))
    el2 = _bdot(al2, feat2, 2, 2)
    er2 = _bdot(feat2, ar2, 2, 2)
    shift2 = _lrelu(jnp.max(el2, axis=2, keepdims=True) + er2)
    ee2 = jnp.exp(_lrelu(el2 + er2) + lct3 - shift2)
    sums2 = _bdot(ee2, feat2_aug, 2, 1)
    den2 = sums2[:, :, OUT1:OUT1 + 1]
    rst2 = jnp.where(den2 > 0.0, sums2[:, :, :OUT1] / den2, 0.0)
    rst2 = (rst2.reshape(B * NP, OUT1) + _dot(h1f, g2_res_ref[0])
            + g2_b_ref[0])
    h2 = jnp.maximum(rst2 * (INV * bn2_g_ref[0]) + bn2_b_ref[0], 0.0)

    rep_ref[:, 0, :, :] = jnp.mean(h2.reshape(B, NP, OUT1), axis=1,
                                   keepdims=True)

    d = _dot(h2, dec_w1_ref[0]) + dec_b1_ref[0]
    d = jnp.maximum(d * (INV * dec_bng_ref[0]) + dec_bnb_ref[0], 0.0)
    recon = _dot(d, dec_w2_ref[0]) + dec_b2_ref[0, 0, 0]
    diff = recon.reshape(B, NP, 1) - h0c
    rloss_ref[:, 0, :, :] = jnp.sum(diff * diff, axis=(1, 2),
                                    keepdims=True) / NP


def _patient_kernel(nf_ref, psrc_ref, pdst_ref, mask_row_ref, mask_col_ref,
                    ft_w_ref, ft_b_ref, ft_bng_ref, ft_bnb_ref,
                    gc_w_ref, gc_b_ref, gbn_g_ref, gbn_b_ref,
                    cl_w1_ref, cl_b1_ref, cl_lng_ref, cl_lnb_ref,
                    cl_w2_ref, cl_b2_ref, rl_ref,
                    logits_ref, avg_ref):
    h = _dot(nf_ref[:, :], ft_w_ref[:, :]) + ft_b_ref[:, :]
    h = jnp.maximum(h * (INV * ft_bng_ref[:, :]) + ft_bnb_ref[:, :], 0.0)

    iota_e = jax.lax.broadcasted_iota(jnp.int32, (NPAT, EPAT), 0)
    ohs = (psrc_ref[:, :] == iota_e).astype(jnp.float32)
    ohd = (pdst_ref[:, :] == iota_e).astype(jnp.float32)
    cp = _dot_t(ohs, ohd)
    ctp = _dot_t(ohd, ohs)
    out_deg = jnp.maximum(jnp.sum(cp, axis=1, keepdims=True), 1.0)
    in_deg = jnp.maximum(jnp.sum(ctp, axis=1, keepdims=True), 1.0)
    dout = jax.lax.rsqrt(out_deg)
    din = jax.lax.rsqrt(in_deg)
    adj = ctp * mask_row_ref[:, :] * mask_col_ref[:, :]

    hsum = h
    for i in range(3):
        agg = _dot(adj, h * dout) * din
        hn = _dot(agg, gc_w_ref[i]) + gc_b_ref[i]
        hn = jnp.maximum(hn * (INV * gbn_g_ref[i]) + gbn_b_ref[i], 0.0)
        h = hn + h
        hsum = hsum + h
    havg = hsum * 0.25

    z = _dot(havg, cl_w1_ref[:, :]) + cl_b1_ref[:, :]
    mu = jnp.mean(z, axis=1, keepdims=True)
    zc = z - mu
    var = jnp.mean(zc * zc, axis=1, keepdims=True)
    z = zc * jax.lax.rsqrt(var + EPS) * cl_lng_ref[:, :] + cl_lnb_ref[:, :]
    z = jnp.maximum(z, 0.0)
    logits_ref[:, :] = _dot(z, cl_w2_ref[:, :]) + cl_b2_ref[:, :]
    avg_ref[:, :] = jnp.reshape(jnp.sum(rl_ref[:, :]) / (B * P), (1, 1))


@jax.jit
def kernel(plane_feat, plane_edge_index, original_features, patient_edge_index,
           mask, g1_fc, g1_al, g1_ar, g1_res, g1_b, bn1_g, bn1_b,
           g2_fc, g2_al, g2_ar, g2_res, g2_b, bn2_g, bn2_b,
           dec_w1, dec_b1, dec_bng, dec_bnb, dec_w2, dec_b2,
           ft_w, ft_b, ft_bng, ft_bnb, gc_w, gc_b, gbn_g, gbn_b,
           cl_w1, cl_b1, cl_lng, cl_lnb, cl_w2, cl_b2):
    f32 = jnp.float32
    pf_col = plane_feat.astype(f32)
    pf_row = pf_col.reshape(B, P, 1, NP)
    src = plane_edge_index[0].astype(jnp.int32).reshape(1, EP)
    dst = plane_edge_index[1].astype(jnp.int32).reshape(1, EP)

    lct = pl.pallas_call(
        _ct_kernel,
        out_shape=jax.ShapeDtypeStruct((NP, NP), f32),
    )(src, dst)

    pspec = lambda blk: pl.BlockSpec(blk, lambda p: (p,) + (0,) * (len(blk) - 1))
    cspec = lambda blk: pl.BlockSpec(blk, lambda p: (0,) * len(blk))
    bspec = lambda blk: pl.BlockSpec(blk, lambda p: (0, p) + (0,) * (len(blk) - 2))

    reps, rloss = pl.pallas_call(
        _plane_kernel,
        grid=(P,),
        in_specs=[
            cspec((NP, NP)),
            bspec((B, 1, NP, 1)),
            bspec((B, 1, 1, NP)),
            pspec((1, 1, HEADS * H1)),
            pspec((1, 1, HEADS * H1)),
            pspec((1, 1, HEADS * H1)),
            pspec((1, 1, HEADS * H1)),
            pspec((1, 1, HEADS * H1)),
            pspec((1, 1, HEADS * H1)),
            pspec((1, 1, HEADS * H1)),
            pspec((1, HEADS * H1, OUT1)),
            pspec((1, 1, OUT1)),
            pspec((1, 1, OUT1)),
            pspec((1, HEADS * H1, OUT1)),
            pspec((1, 1, OUT1)),
            pspec((1, 1, OUT1)),
            pspec((1, 1, OUT1)),
            pspec((1, OUT1, HEADS * H1)),
            pspec((1, 1, HEADS * H1)),
            pspec((1, 1, HEADS * H1)),
            pspec((1, 1, HEADS * H1)),
            pspec((1, HEADS * H1, 1)),
            pspec((1, 1, 1)),
        ],
        out_specs=[
            pl.BlockSpec((B, 1, 1, OUT1), lambda p: (0, p, 0, 0)),
            pl.BlockSpec((B, 1, 1, 1), lambda p: (0, p, 0, 0)),
        ],
        out_shape=[
            jax.ShapeDtypeStruct((B, P, 1, OUT1), f32),
            jax.ShapeDtypeStruct((B, P, 1, 1), f32),
        ],
        compiler_params=pltpu.CompilerParams(
            dimension_semantics=("arbitrary",)),
    )(lct, pf_col, pf_row,
      g1_fc.reshape(P, 1, HEADS * H1), g1_al.reshape(P, 1, HEADS * H1),
      g1_ar.reshape(P, 1, HEADS * H1), g1_res.reshape(P, 1, HEADS * H1),
      g1_b.reshape(P, 1, HEADS * H1), bn1_g.reshape(P, 1, HEADS * H1),
      bn1_b.reshape(P, 1, HEADS * H1),
      g2_fc, g2_al, g2_ar, g2_res,
      g2_b.reshape(P, 1, OUT1), bn2_g.reshape(P, 1, OUT1),
      bn2_b.reshape(P, 1, OUT1),
      dec_w1, dec_b1.reshape(P, 1, HEADS * H1),
      dec_bng.reshape(P, 1, HEADS * H1), dec_bnb.reshape(P, 1, HEADS * H1),
      dec_w2, dec_b2.reshape(P, 1, 1))

    node_features = jnp.concatenate(
        [original_features.astype(f32), reps.reshape(B, P * OUT1)], axis=1)
    psrc = patient_edge_index[0].astype(jnp.int32).reshape(1, EPAT)
    pdst = patient_edge_index[1].astype(jnp.int32).reshape(1, EPAT)
    maskf = mask.astype(f32)

    logits, avg = pl.pallas_call(
        _patient_kernel,
        out_shape=[
            jax.ShapeDtypeStruct((NPAT, 2), f32),
            jax.ShapeDtypeStruct((1, 1), f32),
        ],
    )(node_features, psrc, pdst, maskf.reshape(1, NPAT),
      maskf.reshape(NPAT, 1),
      ft_w, ft_b.reshape(1, NH), ft_bng.reshape(1, NH), ft_bnb.reshape(1, NH),
      gc_w, gc_b.reshape(3, 1, NH), gbn_g.reshape(3, 1, NH),
      gbn_b.reshape(3, 1, NH),
      cl_w1, cl_b1.reshape(1, NH // 2), cl_lng.reshape(1, NH // 2),
      cl_lnb.reshape(1, NH // 2), cl_w2, cl_b2.reshape(1, 2),
      rloss.reshape(B, P))

    return logits, avg.reshape(())


# speedup vs baseline: 1.1127x; 1.1127x over previous
import jax
import jax.numpy as jnp
from jax.experimental import pallas as pl
from jax.experimental.pallas import tpu as pltpu

B = 16; P = 16; NP = 128; EP = 1024
NPAT = 16; EPAT = 80
D_ORIG = 256; H1 = 64; HEADS = 2; OUT1 = 32; NH = 128
EPS = 1e-5
INV = 1.0 / (1.0 + EPS) ** 0.5
NEG = -1e30


def _lrelu(x):
    return jnp.maximum(x, 0.2 * x)


def _dot(a, b):
    return jax.lax.dot_general(a, b, (((1,), (0,)), ((), ())),
                               preferred_element_type=jnp.float32)


def _dot_t(a, b):
    return jax.lax.dot_general(a, b, (((1,), (1,)), ((), ())),
                               preferred_element_type=jnp.float32)


def _bdot(a, b, ca, cb):
    return jax.lax.dot_general(a, b, (((ca,), (cb,)), ((0,), (0,))),
                               preferred_element_type=jnp.float32)


def _ct_kernel(src_ref, dst_ref, lct_ref):
    iota_e = jax.lax.broadcasted_iota(jnp.int32, (NP, EP), 0)
    ohs = (src_ref[:, :] == iota_e).astype(jnp.float32)
    ohd = (dst_ref[:, :] == iota_e).astype(jnp.float32)
    ct = _dot_t(ohd, ohs)
    lct_ref[:, :] = jnp.where(ct > 0.5, jnp.log(jnp.maximum(ct, 0.5)), NEG)


def _plane_kernel(lct_ref, pf_col_ref, pf_row_ref,
                  g1_fc_ref, g1_al_ref, g1_ar_ref, g1_res_ref, g1_b_ref,
                  bn1_g_ref, bn1_b_ref,
                  g2_fc_ref, g2_al_ref, g2_ar_ref, g2_res_ref, g2_b_ref,
                  bn2_g_ref, bn2_b_ref,
                  dec_w1_ref, dec_b1_ref, dec_bng_ref, dec_bnb_ref,
                  dec_w2_ref, dec_b2_ref,
                  rep_ref, rloss_ref):
    lct3 = lct_ref[:, :][None, :, :]

    h0c = pf_col_ref[:, 0, :, :]
    h0r = pf_row_ref[:, 0, :, :]

    fc1 = g1_fc_ref[0]
    al1 = g1_al_ref[0]
    ar1 = g1_ar_ref[0]
    prod_l = fc1 * al1
    prod_r = fc1 * ar1
    cl0 = jnp.sum(prod_l[:, :H1]); cl1 = jnp.sum(prod_l[:, H1:])
    cr0 = jnp.sum(prod_r[:, :H1]); cr1 = jnp.sum(prod_r[:, H1:])
    ones_c = h0c * 0.0 + 1.0
    h0_aug = jnp.concatenate([h0c, ones_c], axis=2)

    def gat1_head(cl, cr):
        q = _lrelu(cl * h0r + cr * h0c) + lct3
        emax = jnp.max(q, axis=2, keepdims=True)
        ee = jnp.exp(q - emax)
        sums = _bdot(ee, h0_aug, 2, 1)
        s = sums[:, :, 0:1] / sums[:, :, 1:2]
        return jnp.where(emax > -1e29, s, 0.0)

    s0 = gat1_head(cl0, cr0)
    s1 = gat1_head(cl1, cr1)
    lane = jax.lax.broadcasted_iota(jnp.int32, (1, HEADS * H1), 1)
    hsel0 = (lane < H1).astype(jnp.float32)
    w3 = jnp.concatenate([fc1 * hsel0, fc1 * (1.0 - hsel0), g1_res_ref[0]],
                         axis=0)
    lhs = jnp.concatenate([s0, s1, h0c], axis=2).reshape(B * NP, 3)
    rst1 = _dot(lhs, w3) + g1_b_ref[0]
    h1f = jnp.maximum(rst1 * (INV * bn1_g_ref[0]) + bn1_b_ref[0], 0.0)

    feat2 = _dot(h1f, g2_fc_ref[0]).reshape(B, NP, OUT1)
    ones2 = feat2[:, :, 0:1] * 0.0 + 1.0
    feat2_aug = jnp.concatenate([feat2, ones2], axis=2)
    al2 = jnp.broadcast_to(g2_al_ref[0][None], (B, 1, OUT1))
    ar2 = jnp.broadcast_to(g2_ar_ref[0][None], (B, 1, OUT1))
    el2 = _bdot(al2, feat2, 2, 2)
    er2 = _bdot(feat2, ar2, 2, 2)
    q2 = _lrelu(el2 + er2) + lct3
    emax2 = jnp.max(q2, axis=2, keepdims=True)
    ee2 = jnp.exp(q2 - emax2)
    sums2 = _bdot(ee2, feat2_aug, 2, 1)
    rst2 = jnp.where(emax2 > -1e29,
                     sums2[:, :, :OUT1] / sums2[:, :, OUT1:OUT1 + 1], 0.0)
    rst2 = (rst2.reshape(B * NP, OUT1) + _dot(h1f, g2_res_ref[0])
            + g2_b_ref[0])
    h2 = jnp.maximum(rst2 * (INV * bn2_g_ref[0]) + bn2_b_ref[0], 0.0)

    rep_ref[:, 0, :, :] = jnp.mean(h2.reshape(B, NP, OUT1), axis=1,
                                   keepdims=True)

    d = _dot(h2, dec_w1_ref[0]) + dec_b1_ref[0]
    d = jnp.maximum(d * (INV * dec_bng_ref[0]) + dec_bnb_ref[0], 0.0)
    recon = _dot(d, dec_w2_ref[0]) + dec_b2_ref[0, 0, 0]
    diff = recon.reshape(B, NP, 1) - h0c
    rloss_ref[:, 0, :, :] = jnp.sum(diff * diff, axis=(1, 2),
                                    keepdims=True) / NP


def _patient_kernel(nf_ref, psrc_ref, pdst_ref, mask_row_ref, mask_col_ref,
                    ft_w_ref, ft_b_ref, ft_bng_ref, ft_bnb_ref,
                    gc_w_ref, gc_b_ref, gbn_g_ref, gbn_b_ref,
                    cl_w1_ref, cl_b1_ref, cl_lng_ref, cl_lnb_ref,
                    cl_w2_ref, cl_b2_ref, rl_ref,
                    logits_ref, avg_ref):
    h = _dot(nf_ref[:, :], ft_w_ref[:, :]) + ft_b_ref[:, :]
    h = jnp.maximum(h * (INV * ft_bng_ref[:, :]) + ft_bnb_ref[:, :], 0.0)

    iota_e = jax.lax.broadcasted_iota(jnp.int32, (NPAT, EPAT), 0)
    ohs = (psrc_ref[:, :] == iota_e).astype(jnp.float32)
    ohd = (pdst_ref[:, :] == iota_e).astype(jnp.float32)
    cp = _dot_t(ohs, ohd)
    ctp = _dot_t(ohd, ohs)
    out_deg = jnp.maximum(jnp.sum(cp, axis=1, keepdims=True), 1.0)
    in_deg = jnp.maximum(jnp.sum(ctp, axis=1, keepdims=True), 1.0)
    dout = jax.lax.rsqrt(out_deg)
    din = jax.lax.rsqrt(in_deg)
    adj = ctp * mask_row_ref[:, :] * mask_col_ref[:, :]

    hsum = h
    for i in range(3):
        agg = _dot(adj, h * dout) * din
        hn = _dot(agg, gc_w_ref[i]) + gc_b_ref[i]
        hn = jnp.maximum(hn * (INV * gbn_g_ref[i]) + gbn_b_ref[i], 0.0)
        h = hn + h
        hsum = hsum + h
    havg = hsum * 0.25

    z = _dot(havg, cl_w1_ref[:, :]) + cl_b1_ref[:, :]
    mu = jnp.mean(z, axis=1, keepdims=True)
    zc = z - mu
    var = jnp.mean(zc * zc, axis=1, keepdims=True)
    z = zc * jax.lax.rsqrt(var + EPS) * cl_lng_ref[:, :] + cl_lnb_ref[:, :]
    z = jnp.maximum(z, 0.0)
    logits_ref[:, :] = _dot(z, cl_w2_ref[:, :]) + cl_b2_ref[:, :]
    avg_ref[:, :] = jnp.reshape(jnp.sum(rl_ref[:, :]) / (B * P), (1, 1))


@jax.jit
def kernel(plane_feat, plane_edge_index, original_features, patient_edge_index,
           mask, g1_fc, g1_al, g1_ar, g1_res, g1_b, bn1_g, bn1_b,
           g2_fc, g2_al, g2_ar, g2_res, g2_b, bn2_g, bn2_b,
           dec_w1, dec_b1, dec_bng, dec_bnb, dec_w2, dec_b2,
           ft_w, ft_b, ft_bng, ft_bnb, gc_w, gc_b, gbn_g, gbn_b,
           cl_w1, cl_b1, cl_lng, cl_lnb, cl_w2, cl_b2):
    f32 = jnp.float32
    pf_col = plane_feat.astype(f32)
    pf_row = pf_col.reshape(B, P, 1, NP)
    src = plane_edge_index[0].astype(jnp.int32).reshape(1, EP)
    dst = plane_edge_index[1].astype(jnp.int32).reshape(1, EP)

    lct = pl.pallas_call(
        _ct_kernel,
        out_shape=jax.ShapeDtypeStruct((NP, NP), f32),
    )(src, dst)

    pspec = lambda blk: pl.BlockSpec(blk, lambda p: (p,) + (0,) * (len(blk) - 1))
    cspec = lambda blk: pl.BlockSpec(blk, lambda p: (0,) * len(blk))
    bspec = lambda blk: pl.BlockSpec(blk, lambda p: (0, p) + (0,) * (len(blk) - 2))

    reps, rloss = pl.pallas_call(
        _plane_kernel,
        grid=(P,),
        in_specs=[
            cspec((NP, NP)),
            bspec((B, 1, NP, 1)),
            bspec((B, 1, 1, NP)),
            pspec((1, 1, HEADS * H1)),
            pspec((1, 1, HEADS * H1)),
            pspec((1, 1, HEADS * H1)),
            pspec((1, 1, HEADS * H1)),
            pspec((1, 1, HEADS * H1)),
            pspec((1, 1, HEADS * H1)),
            pspec((1, 1, HEADS * H1)),
            pspec((1, HEADS * H1, OUT1)),
            pspec((1, 1, OUT1)),
            pspec((1, 1, OUT1)),
            pspec((1, HEADS * H1, OUT1)),
            pspec((1, 1, OUT1)),
            pspec((1, 1, OUT1)),
            pspec((1, 1, OUT1)),
            pspec((1, OUT1, HEADS * H1)),
            pspec((1, 1, HEADS * H1)),
            pspec((1, 1, HEADS * H1)),
            pspec((1, 1, HEADS * H1)),
            pspec((1, HEADS * H1, 1)),
            pspec((1, 1, 1)),
        ],
        out_specs=[
            pl.BlockSpec((B, 1, 1, OUT1), lambda p: (0, p, 0, 0)),
            pl.BlockSpec((B, 1, 1, 1), lambda p: (0, p, 0, 0)),
        ],
        out_shape=[
            jax.ShapeDtypeStruct((B, P, 1, OUT1), f32),
            jax.ShapeDtypeStruct((B, P, 1, 1), f32),
        ],
        compiler_params=pltpu.CompilerParams(
            dimension_semantics=("parallel",)),
    )(lct, pf_col, pf_row,
      g1_fc.reshape(P, 1, HEADS * H1), g1_al.reshape(P, 1, HEADS * H1),
      g1_ar.reshape(P, 1, HEADS * H1), g1_res.reshape(P, 1, HEADS * H1),
      g1_b.reshape(P, 1, HEADS * H1), bn1_g.reshape(P, 1, HEADS * H1),
      bn1_b.reshape(P, 1, HEADS * H1),
      g2_fc, g2_al, g2_ar, g2_res,
      g2_b.reshape(P, 1, OUT1), bn2_g.reshape(P, 1, OUT1),
      bn2_b.reshape(P, 1, OUT1),
      dec_w1, dec_b1.reshape(P, 1, HEADS * H1),
      dec_bng.reshape(P, 1, HEADS * H1), dec_bnb.reshape(P, 1, HEADS * H1),
      dec_w2, dec_b2.reshape(P, 1, 1))

    node_features = jnp.concatenate(
        [original_features.astype(f32), reps.reshape(B, P * OUT1)], axis=1)
    psrc = patient_edge_index[0].astype(jnp.int32).reshape(1, EPAT)
    pdst = patient_edge_index[1].astype(jnp.int32).reshape(1, EPAT)
    maskf = mask.astype(f32)

    logits, avg = pl.pallas_call(
        _patient_kernel,
        out_shape=[
            jax.ShapeDtypeStruct((NPAT, 2), f32),
            jax.ShapeDtypeStruct((1, 1), f32),
        ],
    )(node_features, psrc, pdst, maskf.reshape(1, NPAT),
      maskf.reshape(NPAT, 1),
      ft_w, ft_b.reshape(1, NH), ft_bng.reshape(1, NH), ft_bnb.reshape(1, NH),
      gc_w, gc_b.reshape(3, 1, NH), gbn_g.reshape(3, 1, NH),
      gbn_b.reshape(3, 1, NH),
      cl_w1, cl_b1.reshape(1, NH // 2), cl_lng.reshape(1, NH // 2),
      cl_lnb.reshape(1, NH // 2), cl_w2, cl_b2.reshape(1, 2),
      rloss.reshape(B, P))

    return logits, avg.reshape(())


# speedup vs baseline: 1.5033x; 1.3510x over previous
import jax
import jax.numpy as jnp
from jax.experimental import pallas as pl
from jax.experimental.pallas import tpu as pltpu

B = 16; P = 16; NP = 128; EP = 1024
NPAT = 16; EPAT = 80
D_ORIG = 256; H1 = 64; HEADS = 2; OUT1 = 32; NH = 128
EPS = 1e-5
INV = 1.0 / (1.0 + EPS) ** 0.5
NEG = -1e30


def _lrelu(x):
    return jnp.maximum(x, 0.2 * x)


def _dot(a, b):
    return jax.lax.dot_general(a, b, (((1,), (0,)), ((), ())),
                               preferred_element_type=jnp.float32)


def _dot_t(a, b):
    return jax.lax.dot_general(a, b, (((1,), (1,)), ((), ())),
                               preferred_element_type=jnp.float32)


def _bdot(a, b, ca, cb):
    return jax.lax.dot_general(a, b, (((ca,), (cb,)), ((0,), (0,))),
                               preferred_element_type=jnp.float32)


def _bc(x):
    return jnp.broadcast_to(x[None], (B,) + x.shape)


def _ct_kernel(src_ref, dst_ref, lct_ref):
    iota_e = jax.lax.broadcasted_iota(jnp.int32, (NP, EP), 0)
    ohs = (src_ref[:, :] == iota_e).astype(jnp.float32)
    ohd = (dst_ref[:, :] == iota_e).astype(jnp.float32)
    ct = _dot_t(ohs, ohd)
    lct_ref[:, :] = jnp.where(ct > 0.5, jnp.log(jnp.maximum(ct, 0.5)), NEG)


def _plane_kernel(lct_ref, pf_col_ref, pf_row_ref,
                  g1_fc_ref, g1_al_ref, g1_ar_ref, g1_res_ref, g1_b_ref,
                  bn1_g_ref, bn1_b_ref,
                  g2_fc_ref, g2_al_ref, g2_ar_ref, g2_res_ref, g2_b_ref,
                  bn2_g_ref, bn2_b_ref,
                  dec_w1_ref, dec_b1_ref, dec_bng_ref, dec_bnb_ref,
                  dec_w2_ref, dec_b2_ref,
                  rep_ref, rloss_ref):
    lct3 = lct_ref[:, :][None, :, :]

    h0c = pf_col_ref[:, 0, :, :]
    h0r = pf_row_ref[:, 0, :, :]
    onesr = h0r * 0.0 + 1.0
    onesc = h0c * 0.0 + 1.0

    onesrow = jnp.full((1, NP), 1.0, jnp.float32)
    h0b = _dot(h0c.reshape(B * NP, 1), onesrow).reshape(B, NP, NP)

    fc1 = g1_fc_ref[0]
    al1 = g1_al_ref[0]
    ar1 = g1_ar_ref[0]
    prod_l = fc1 * al1
    prod_r = fc1 * ar1
    cl0 = jnp.sum(prod_l[:, :H1]); cl1 = jnp.sum(prod_l[:, H1:])
    cr0 = jnp.sum(prod_r[:, :H1]); cr1 = jnp.sum(prod_r[:, H1:])
    h0_aug = jnp.concatenate([h0c, onesc], axis=2)

    def gat1_head(cl, cr):
        q = _lrelu(cl * h0b + cr * h0r) + lct3
        emax = jnp.max(q, axis=1, keepdims=True)
        ee = jnp.exp(q - emax)
        sums = _bdot(h0_aug, ee, 1, 1)
        s = sums[:, 0:1, :] / sums[:, 1:2, :]
        return jnp.where(emax > -1e29, s, 0.0)

    s0 = gat1_head(cl0, cr0)
    s1 = gat1_head(cl1, cr1)
    lane = jax.lax.broadcasted_iota(jnp.int32, (1, HEADS * H1), 1)
    hsel0 = (lane < H1).astype(jnp.float32)
    bn1s = INV * bn1_g_ref[0]
    w4 = jnp.concatenate([fc1 * hsel0 * bn1s, fc1 * (1.0 - hsel0) * bn1s,
                          g1_res_ref[0] * bn1s,
                          g1_b_ref[0] * bn1s + bn1_b_ref[0]], axis=0)
    lhs4 = jnp.concatenate([s0, s1, h0r, onesr], axis=1)
    h1ft = jnp.maximum(_bdot(_bc(w4), lhs4, 1, 1), 0.0)

    feat2t = _bdot(_bc(g2_fc_ref[0]), h1ft, 1, 1)
    onescol = jnp.full((NP, 1), 1.0, jnp.float32)
    w2t = _dot(onescol, g2_al_ref[0])
    el2b = _bdot(feat2t, _bc(w2t), 1, 2)
    er2 = _bdot(_bc(g2_ar_ref[0]), feat2t, 2, 1)
    q2 = _lrelu(el2b + er2) + lct3
    emax2 = jnp.max(q2, axis=1, keepdims=True)
    ee2 = jnp.exp(q2 - emax2)
    bn2s = INV * bn2_g_ref[0]
    io = jax.lax.broadcasted_iota(jnp.int32, (OUT1, OUT1), 0)
    it = jax.lax.broadcasted_iota(jnp.int32, (OUT1, OUT1), 1)
    dscale = jnp.where(io == it, 1.0, 0.0) * bn2s
    feat2s = _bdot(_bc(dscale), feat2t, 1, 1)
    vals = jnp.concatenate([feat2s, onesr], axis=1)
    sums2 = _bdot(vals, ee2, 2, 1)
    att2 = jnp.where(emax2 > -1e29,
                     sums2[:, :OUT1, :] / sums2[:, OUT1:OUT1 + 1, :], 0.0)
    g2aug = jnp.concatenate(
        [g2_res_ref[0] * bn2s,
         g2_b_ref[0] * bn2s + bn2_b_ref[0]], axis=0)
    h1faug = jnp.concatenate([h1ft, onesr], axis=1)
    h2t = jnp.maximum(att2 + _bdot(_bc(g2aug), h1faug, 1, 1), 0.0)

    rep_ref[:, 0, :, :] = _bdot(h2t, _bc(onescol), 2, 1) * (1.0 / NP)

    decs = INV * dec_bng_ref[0]
    w1aug = jnp.concatenate(
        [dec_w1_ref[0] * decs,
         dec_b1_ref[0] * decs + dec_bnb_ref[0]], axis=0)
    h2aug = jnp.concatenate([h2t, onesr], axis=1)
    dt = jnp.maximum(_bdot(_bc(w1aug), h2aug, 1, 1), 0.0)
    recon = _bdot(_bc(dec_w2_ref[0]), dt, 1, 1) + dec_b2_ref[0, 0, 0]
    diff = recon - h0r
    rloss_ref[:, 0, :, :] = _bdot(diff * diff, _bc(onescol), 2, 1) / NP


def _patient_kernel(nf_ref, psrc_ref, pdst_ref, mask_row_ref, mask_col_ref,
                    ft_w_ref, ft_b_ref, ft_bng_ref, ft_bnb_ref,
                    gc_w_ref, gc_b_ref, gbn_g_ref, gbn_b_ref,
                    cl_w1_ref, cl_b1_ref, cl_lng_ref, cl_lnb_ref,
                    cl_w2_ref, cl_b2_ref, rl_ref,
                    logits_ref, avg_ref):
    h = _dot(nf_ref[:, :], ft_w_ref[:, :]) + ft_b_ref[:, :]
    h = jnp.maximum(h * (INV * ft_bng_ref[:, :]) + ft_bnb_ref[:, :], 0.0)

    iota_e = jax.lax.broadcasted_iota(jnp.int32, (NPAT, EPAT), 0)
    ohs = (psrc_ref[:, :] == iota_e).astype(jnp.float32)
    ohd = (pdst_ref[:, :] == iota_e).astype(jnp.float32)
    cp = _dot_t(ohs, ohd)
    ctp = _dot_t(ohd, ohs)
    out_deg = jnp.maximum(jnp.sum(cp, axis=1, keepdims=True), 1.0)
    in_deg = jnp.maximum(jnp.sum(ctp, axis=1, keepdims=True), 1.0)
    dout = jax.lax.rsqrt(out_deg)
    din = jax.lax.rsqrt(in_deg)
    adj = ctp * mask_row_ref[:, :] * mask_col_ref[:, :]

    hsum = h
    for i in range(3):
        agg = _dot(adj, h * dout) * din
        hn = _dot(agg, gc_w_ref[i]) + gc_b_ref[i]
        hn = jnp.maximum(hn * (INV * gbn_g_ref[i]) + gbn_b_ref[i], 0.0)
        h = hn + h
        hsum = hsum + h
    havg = hsum * 0.25

    z = _dot(havg, cl_w1_ref[:, :]) + cl_b1_ref[:, :]
    mu = jnp.mean(z, axis=1, keepdims=True)
    zc = z - mu
    var = jnp.mean(zc * zc, axis=1, keepdims=True)
    z = zc * jax.lax.rsqrt(var + EPS) * cl_lng_ref[:, :] + cl_lnb_ref[:, :]
    z = jnp.maximum(z, 0.0)
    logits_ref[:, :] = _dot(z, cl_w2_ref[:, :]) + cl_b2_ref[:, :]
    avg_ref[:, :] = jnp.reshape(jnp.sum(rl_ref[:, :]) / (B * P), (1, 1))


@jax.jit
def kernel(plane_feat, plane_edge_index, original_features, patient_edge_index,
           mask, g1_fc, g1_al, g1_ar, g1_res, g1_b, bn1_g, bn1_b,
           g2_fc, g2_al, g2_ar, g2_res, g2_b, bn2_g, bn2_b,
           dec_w1, dec_b1, dec_bng, dec_bnb, dec_w2, dec_b2,
           ft_w, ft_b, ft_bng, ft_bnb, gc_w, gc_b, gbn_g, gbn_b,
           cl_w1, cl_b1, cl_lng, cl_lnb, cl_w2, cl_b2):
    f32 = jnp.float32
    pf_col = plane_feat.astype(f32)
    pf_row = pf_col.reshape(B, P, 1, NP)
    src = plane_edge_index[0].astype(jnp.int32).reshape(1, EP)
    dst = plane_edge_index[1].astype(jnp.int32).reshape(1, EP)

    lct = pl.pallas_call(
        _ct_kernel,
        out_shape=jax.ShapeDtypeStruct((NP, NP), f32),
    )(src, dst)

    pspec = lambda blk: pl.BlockSpec(blk, lambda p: (p,) + (0,) * (len(blk) - 1))
    cspec = lambda blk: pl.BlockSpec(blk, lambda p: (0,) * len(blk))
    bspec = lambda blk: pl.BlockSpec(blk, lambda p: (0, p) + (0,) * (len(blk) - 2))

    reps, rloss = pl.pallas_call(
        _plane_kernel,
        grid=(P,),
        in_specs=[
            cspec((NP, NP)),
            bspec((B, 1, NP, 1)),
            bspec((B, 1, 1, NP)),
            pspec((1, 1, HEADS * H1)),
            pspec((1, 1, HEADS * H1)),
            pspec((1, 1, HEADS * H1)),
            pspec((1, 1, HEADS * H1)),
            pspec((1, 1, HEADS * H1)),
            pspec((1, 1, HEADS * H1)),
            pspec((1, 1, HEADS * H1)),
            pspec((1, HEADS * H1, OUT1)),
            pspec((1, 1, OUT1)),
            pspec((1, 1, OUT1)),
            pspec((1, HEADS * H1, OUT1)),
            pspec((1, 1, OUT1)),
            pspec((1, 1, OUT1)),
            pspec((1, 1, OUT1)),
            pspec((1, OUT1, HEADS * H1)),
            pspec((1, 1, HEADS * H1)),
            pspec((1, 1, HEADS * H1)),
            pspec((1, 1, HEADS * H1)),
            pspec((1, HEADS * H1, 1)),
            pspec((1, 1, 1)),
        ],
        out_specs=[
            pl.BlockSpec((B, 1, OUT1, 1), lambda p: (0, p, 0, 0)),
            pl.BlockSpec((B, 1, 1, 1), lambda p: (0, p, 0, 0)),
        ],
        out_shape=[
            jax.ShapeDtypeStruct((B, P, OUT1, 1), f32),
            jax.ShapeDtypeStruct((B, P, 1, 1), f32),
        ],
        compiler_params=pltpu.CompilerParams(
            dimension_semantics=("parallel",)),
    )(lct, pf_col, pf_row,
      g1_fc.reshape(P, 1, HEADS * H1), g1_al.reshape(P, 1, HEADS * H1),
      g1_ar.reshape(P, 1, HEADS * H1), g1_res.reshape(P, 1, HEADS * H1),
      g1_b.reshape(P, 1, HEADS * H1), bn1_g.reshape(P, 1, HEADS * H1),
      bn1_b.reshape(P, 1, HEADS * H1),
      g2_fc, g2_al, g2_ar, g2_res,
      g2_b.reshape(P, 1, OUT1), bn2_g.reshape(P, 1, OUT1),
      bn2_b.reshape(P, 1, OUT1),
      dec_w1, dec_b1.reshape(P, 1, HEADS * H1),
      dec_bng.reshape(P, 1, HEADS * H1), dec_bnb.reshape(P, 1, HEADS * H1),
      dec_w2, dec_b2.reshape(P, 1, 1))

    node_features = jnp.concatenate(
        [original_features.astype(f32), reps.reshape(B, P * OUT1)], axis=1)
    psrc = patient_edge_index[0].astype(jnp.int32).reshape(1, EPAT)
    pdst = patient_edge_index[1].astype(jnp.int32).reshape(1, EPAT)
    maskf = mask.astype(f32)

    logits, avg = pl.pallas_call(
        _patient_kernel,
        out_shape=[
            jax.ShapeDtypeStruct((NPAT, 2), f32),
            jax.ShapeDtypeStruct((1, 1), f32),
        ],
    )(node_features, psrc, pdst, maskf.reshape(1, NPAT),
      maskf.reshape(NPAT, 1),
      ft_w, ft_b.reshape(1, NH), ft_bng.reshape(1, NH), ft_bnb.reshape(1, NH),
      gc_w, gc_b.reshape(3, 1, NH), gbn_g.reshape(3, 1, NH),
      gbn_b.reshape(3, 1, NH),
      cl_w1, cl_b1.reshape(1, NH // 2), cl_lng.reshape(1, NH // 2),
      cl_lnb.reshape(1, NH // 2), cl_w2, cl_b2.reshape(1, 2),
      rloss.reshape(B, P))

    return logits, avg.reshape(())
